# baseline (speedup 1.0000x reference)
import functools

import jax
import jax.numpy as jnp
from jax import lax
from jax.experimental import pallas as pl
from jax.experimental.pallas import tpu as pltpu
from jax.experimental.pallas import tpu_sc as plsc

_NC = 2
_NS = 16
_NW = _NC * _NS
_LANES = 16
_VPAD = 1024


def _score_table_body(emb_ref, w_ref, out_ref, *, inv_len):
    out_ref[...] = jnp.zeros_like(out_ref)
    e = emb_ref[...]
    w = w_ref[...]
    out_ref[0 : e.shape[0], :] = jnp.sum(e * w, axis=1, keepdims=True) * inv_len


def _make_sc_pool(B, SW):
    R = B // _NW
    C = 128
    NCH = R // C
    U = 4
    mesh = plsc.VectorSubcoreMesh(core_axis_name="c", subcore_axis_name="s")

    @functools.partial(
        pl.kernel,
        mesh=mesh,
        out_type=jax.ShapeDtypeStruct((B,), jnp.float32),
        scratch_types=[
            pltpu.VMEM((C, SW), jnp.int32),
            pltpu.VMEM((C, SW), jnp.int32),
            pltpu.VMEM((_VPAD,), jnp.float32),
            pltpu.VMEM((_LANES,), jnp.float32),
            pltpu.VMEM((R,), jnp.float32),
            pltpu.SemaphoreType.DMA,
            pltpu.SemaphoreType.DMA,
        ],
        compiler_params=pltpu.CompilerParams(needs_layout_passes=False),
    )
    def sc_pool(x_hbm, s_hbm, b_hbm, out_hbm, x_v0, x_v1, s_v, b_v, out_v,
                sem0, sem1):
        wid = lax.axis_index("s") * _NC + lax.axis_index("c")
        base = wid * R
        bufs = (x_v0, x_v1)
        sems = (sem0, sem1)
        pending = {0: pltpu.async_copy(x_hbm.at[pl.ds(base, C)], x_v0, sem0)}
        pltpu.sync_copy(s_hbm, s_v)
        pltpu.sync_copy(b_hbm, b_v)
        lane = lax.iota(jnp.int32, _LANES)
        bv = b_v[...]

        for k in range(NCH):
            if k + 1 < NCH:
                pending[k + 1] = pltpu.async_copy(
                    x_hbm.at[pl.ds(base + (k + 1) * C, C)],
                    bufs[(k + 1) % 2], sems[(k + 1) % 2])
            pending.pop(k).wait()
            x_v = bufs[k % 2]

            def group(g, _, x_v=x_v, k=k):
                rows = lane + g * _LANES

                def step(t, carry):
                    acc0, acc1, col = carry
                    parts0 = []
                    parts1 = []
                    for u in range(U):
                        xw = plsc.load_gather(x_v, [rows, col + u])
                        loi = xw & 0xFFFF
                        hii = lax.shift_right_logical(xw, 16)
                        parts0.append(plsc.load_gather(s_v, [loi]))
                        parts1.append(plsc.load_gather(s_v, [hii]))
                    acc0 = acc0 + ((parts0[0] + parts0[1])
                                   + (parts0[2] + parts0[3]))
                    acc1 = acc1 + ((parts1[0] + parts1[1])
                                   + (parts1[2] + parts1[3]))
                    return acc0, acc1, col + U

                zero = jnp.zeros((_LANES,), jnp.float32)
                col0 = jnp.zeros((_LANES,), jnp.int32)
                acc0, acc1, _ = lax.fori_loop(0, SW // U, step,
                                              (zero, zero, col0))
                z = acc0 + acc1 + bv
                out_v[pl.ds(k * C + g * _LANES, _LANES)] = (
                    1.0 / (1.0 + jnp.exp(-z)))
                return 0

            lax.fori_loop(0, C // _LANES, group, 0)

        pltpu.sync_copy(out_v, out_hbm.at[pl.ds(base, R)])

    return sc_pool


def kernel(x, emb, W, b):
    B, S = x.shape
    V, D = emb.shape
    s2d = pl.pallas_call(
        functools.partial(_score_table_body, inv_len=1.0 / S),
        out_shape=jax.ShapeDtypeStruct((_VPAD, 1), jnp.float32),
    )(emb, W)
    s_flat = s2d.reshape(_VPAD)
    b16 = jnp.broadcast_to(b.astype(jnp.float32), (_LANES,))
    xp = lax.bitcast_convert_type(
        x.astype(jnp.uint16).reshape(B, S // 2, 2), jnp.int32)
    out_flat = _make_sc_pool(B, S // 2)(xp, s_flat, b16)
    return out_flat.reshape(B, 1)


# speedup vs baseline: 1.7746x; 1.7746x over previous
import functools

import jax
import jax.numpy as jnp
from jax import lax
from jax.experimental import pallas as pl
from jax.experimental.pallas import tpu as pltpu
from jax.experimental.pallas import tpu_sc as plsc

_NC = 2
_NS = 16
_NW = _NC * _NS
_LANES = 16
_VPAD = 1024


def _score_table_body(emb_ref, w_ref, out_ref, *, inv_len):
    out_ref[...] = jnp.zeros_like(out_ref)
    e = emb_ref[...]
    w = w_ref[...]
    out_ref[0 : e.shape[0], :] = jnp.sum(e * w, axis=1, keepdims=True) * inv_len


def _make_sc_pool(B, SW):
    R = B // _NW
    C = 128
    NCH = R // C
    U = 4
    mesh = plsc.VectorSubcoreMesh(core_axis_name="c", subcore_axis_name="s")

    @functools.partial(
        pl.kernel,
        mesh=mesh,
        out_type=jax.ShapeDtypeStruct((B,), jnp.float32),
        scratch_types=[
            pltpu.VMEM((C, SW), jnp.int32),
            pltpu.VMEM((C, SW), jnp.int32),
            pltpu.VMEM((_VPAD,), jnp.float32),
            pltpu.VMEM((_LANES,), jnp.float32),
            pltpu.VMEM((R,), jnp.float32),
            pltpu.SemaphoreType.DMA,
            pltpu.SemaphoreType.DMA,
        ],
        compiler_params=pltpu.CompilerParams(needs_layout_passes=False),
    )
    def sc_pool(x_hbm, s_hbm, b_hbm, out_hbm, x_v0, x_v1, s_v, b_v, out_v,
                sem0, sem1):
        wid = lax.axis_index("s") * _NC + lax.axis_index("c")
        base = wid * R
        bufs = (x_v0, x_v1)
        sems = (sem0, sem1)
        pending = {0: pltpu.async_copy(x_hbm.at[pl.ds(base, C)], x_v0, sem0)}
        pltpu.sync_copy(s_hbm, s_v)
        pltpu.sync_copy(b_hbm, b_v)
        lane = lax.iota(jnp.int32, _LANES)
        bv = b_v[...]

        for k in range(NCH):
            if k + 1 < NCH:
                pending[k + 1] = pltpu.async_copy(
                    x_hbm.at[pl.ds(base + (k + 1) * C, C)],
                    bufs[(k + 1) % 2], sems[(k + 1) % 2])
            pending.pop(k).wait()
            x_v = bufs[k % 2]

            def group(g, _, x_v=x_v, k=k):
                rows = lane + g * _LANES

                def step(t, carry):
                    acc0, acc1, col = carry
                    parts0 = []
                    parts1 = []
                    for u in range(U):
                        xw = plsc.load_gather(x_v, [rows, col + u])
                        loi = xw & 0xFFFF
                        hii = lax.shift_right_logical(xw, 16)
                        parts0.append(plsc.load_gather(s_v, [loi]))
                        parts1.append(plsc.load_gather(s_v, [hii]))
                    acc0 = acc0 + ((parts0[0] + parts0[1])
                                   + (parts0[2] + parts0[3]))
                    acc1 = acc1 + ((parts1[0] + parts1[1])
                                   + (parts1[2] + parts1[3]))
                    return acc0, acc1, col + U

                zero = jnp.zeros((_LANES,), jnp.float32)
                col0 = jnp.zeros((_LANES,), jnp.int32)
                acc0, acc1, _ = lax.fori_loop(0, SW // U, step,
                                              (zero, zero, col0))
                z = acc0 + acc1 + bv
                out_v[pl.ds(k * C + g * _LANES, _LANES)] = (
                    1.0 / (1.0 + jnp.exp(-z)))
                return 0

            lax.fori_loop(0, C // _LANES, group, 0)

        pltpu.sync_copy(out_v, out_hbm.at[pl.ds(base, R)])

    return sc_pool


def kernel(x, emb, W, b):
    B, S = x.shape
    V, D = emb.shape
    s2d = pl.pallas_call(
        functools.partial(_score_table_body, inv_len=1.0 / S),
        out_shape=jax.ShapeDtypeStruct((_VPAD, 1), jnp.float32),
    )(emb, W)
    s_flat = s2d.reshape(_VPAD)
    b16 = jnp.broadcast_to(b.astype(jnp.float32), (_LANES,))
    xi = x.astype(jnp.int32)
    xp = xi[:, : S // 2] | (xi[:, S // 2 :] << 16)
    out_flat = _make_sc_pool(B, S // 2)(xp, s_flat, b16)
    return out_flat.reshape(B, 1)
